# baseline (speedup 1.0000x reference)
import functools

import jax
import jax.numpy as jnp
from jax import lax
from jax.experimental import pallas as pl
from jax.experimental.pallas import tpu as pltpu
from jax.experimental.pallas import tpu_sc as plsc

N = 10000
D = 128
E = 320000
NC = 2
NS = 16
NW = NC * NS
B = 128
TPW = 80
ACC_ROWS = 10240
RPT = ACC_ROWS // NS
CNT_W = 16
CF = 2
FW = D // CF
BM = 2000

_MESH = plsc.VectorSubcoreMesh(core_axis_name="c", subcore_axis_name="s")


def _sc_body(with_counts, NBUF, *refs):
    if with_counts:
        (table, src3, dst3, out_p, out_c, src_v, dst_v, *rest) = refs
        rows = rest[:NBUF]
        obuf, zcnt, acc, cnt = rest[NBUF:NBUF + 4]
        sems = rest[NBUF + 4:]
        gs, ss, cs = sems[:NBUF], sems[NBUF:2 * NBUF], sems[2 * NBUF:]
    else:
        table, src3, dst3, out_p, src_v, dst_v, *rest = refs
        rows = rest[:NBUF]
        acc = rest[NBUF]
        sems = rest[NBUF + 1:]
        gs, ss = sems[:NBUF], sems[NBUF:2 * NBUF]
    cid = lax.axis_index("c")
    sid = lax.axis_index("s")
    wid = sid * NC + cid

    pltpu.sync_copy(src3.at[wid], src_v)
    pltpu.sync_copy(dst3.at[wid], dst_v)

    zero16 = jnp.zeros((16,), jnp.float32)
    if with_counts:
        one16 = jnp.ones((16,), jnp.float32)

        def _orow(r, _):
            obuf[r, pl.ds(0, 16)] = one16
            zcnt[r, pl.ds(0, 16)] = zero16
            return 0

        lax.fori_loop(0, B, _orow, 0)

    for cf in range(CF):
        first = cf == 0
        def _zrow(r, _):
            for c8 in range(FW // 16):
                rows[0][r, pl.ds(c8 * 16, 16)] = zero16
            return 0

        lax.fori_loop(0, B, _zrow, 0)
        for k in range(RPT // B):
            pltpu.sync_copy(rows[0], acc.at[pl.ds(sid * RPT + k * B, B)])
            if with_counts and first:
                pltpu.sync_copy(zcnt, cnt.at[pl.ds(sid * RPT + k * B, B)])
        plsc.subcore_barrier()

        wc = with_counts and first

        def _wait_gather(j, b):
            pltpu.make_async_copy(table.at[src_v.at[j]], rows[b],
                                  gs[b]).wait()

        def _issue_scatter(j, b):
            pltpu.async_copy(rows[b], acc.at[dst_v.at[j]], ss[b], add=True)
            if wc:
                pltpu.async_copy(obuf, cnt.at[dst_v.at[j]], cs[b], add=True)

        def _wait_scatter(j, b):
            pltpu.make_async_copy(rows[b], acc.at[dst_v.at[j]], ss[b]).wait()
            if wc:
                pltpu.make_async_copy(obuf, cnt.at[dst_v.at[j]],
                                      cs[b]).wait()

        for b in range(NBUF):
            pltpu.async_copy(table.at[src_v.at[b]], rows[b], gs[b])

        def _super(t, _):
            for b in range(NBUF):
                j = t * NBUF + b
                _wait_gather(j, b)
                _issue_scatter(j, b)
            for b in range(NBUF):
                j = t * NBUF + b
                _wait_scatter(j, b)
                pltpu.async_copy(table.at[src_v.at[j + NBUF]], rows[b],
                                 gs[b])
            return 0

        lax.fori_loop(0, TPW // NBUF - 1, _super, 0)

        last = (TPW // NBUF - 1) * NBUF
        for b in range(NBUF):
            _wait_gather(last + b, b)
            _issue_scatter(last + b, b)
        for b in range(NBUF):
            _wait_scatter(last + b, b)
        plsc.subcore_barrier()

        pltpu.sync_copy(acc.at[pl.ds(sid * RPT, RPT)],
                        out_p.at[cid, pl.ds(sid * RPT, RPT),
                                 pl.ds(cf * FW, FW)])
        if with_counts and first:
            pltpu.sync_copy(cnt.at[pl.ds(sid * RPT, RPT)],
                            out_c.at[cid, pl.ds(sid * RPT, RPT)])

        if first:
            one16i = jnp.ones((16,), jnp.int32)

            def _inc(r, _):
                for c8 in range(B // 16):
                    sl = (r, pl.ds(c8 * 16, 16))
                    src_v[sl] = src_v[sl] + one16i
                return 0

            lax.fori_loop(0, TPW, _inc, 0)


def _make_segsum(with_counts, NBUF):
    out_type = [jax.ShapeDtypeStruct((NC, ACC_ROWS, D), jnp.float32)]
    scratch = [
        pltpu.VMEM((TPW, B), jnp.int32),
        pltpu.VMEM((TPW, B), jnp.int32),
    ]
    scratch += [pltpu.VMEM((B, FW), jnp.float32)] * NBUF
    if with_counts:
        out_type.append(jax.ShapeDtypeStruct((NC, ACC_ROWS, CNT_W),
                                             jnp.float32))
        scratch += [
            pltpu.VMEM((B, CNT_W), jnp.float32),
            pltpu.VMEM((B, CNT_W), jnp.float32),
        ]
    scratch.append(pltpu.VMEM_SHARED((ACC_ROWS, FW), jnp.float32))
    if with_counts:
        scratch.append(pltpu.VMEM_SHARED((ACC_ROWS, CNT_W), jnp.float32))
    n_sems = (3 if with_counts else 2) * NBUF
    scratch += [pltpu.SemaphoreType.DMA] * n_sems
    return pl.kernel(
        functools.partial(_sc_body, with_counts, NBUF),
        out_type=tuple(out_type) if with_counts else out_type[0],
        mesh=_MESH,
        scratch_types=scratch,
        compiler_params=pltpu.CompilerParams(use_tc_tiling_on_sc=False),
    )


_segsum_counts = _make_segsum(True, 4)
_segsum_plain = _make_segsum(False, 8)


def _root_body(x_ref, wr_ref, bl_ref, o_ref):
    o_ref[...] = (jnp.dot(x_ref[...], wr_ref[...],
                          preferred_element_type=jnp.float32) + bl_ref[...])


_mm_root = pl.pallas_call(
    _root_body,
    grid=(N // BM,),
    in_specs=[
        pl.BlockSpec((BM, D), lambda i: (i, 0)),
        pl.BlockSpec((D, D), lambda i: (0, 0)),
        pl.BlockSpec((1, D), lambda i: (0, 0)),
    ],
    out_specs=pl.BlockSpec((BM, D), lambda i: (i, 0)),
    out_shape=jax.ShapeDtypeStruct((N, D), jnp.float32),
)


def _mm_body(relu, p_ref, c_ref, r_ref, wl_ref, o_ref):
    p = p_ref[0] + p_ref[1]
    cnt = (c_ref[0] + c_ref[1])[:, 0:1]
    mean = p / jnp.maximum(cnt, 1.0)
    y = (jnp.dot(mean, wl_ref[...], preferred_element_type=jnp.float32)
         + r_ref[...])
    if relu:
        y = jnp.maximum(y, 0.0)
    o_ref[...] = y


def _make_mm(relu):
    grid = (N // BM,)
    return pl.pallas_call(
        functools.partial(_mm_body, relu),
        grid=grid,
        in_specs=[
            pl.BlockSpec((NC, BM, D), lambda i: (0, i, 0)),
            pl.BlockSpec((NC, BM, CNT_W), lambda i: (0, i, 0)),
            pl.BlockSpec((BM, D), lambda i: (i, 0)),
            pl.BlockSpec((D, D), lambda i: (0, 0)),
        ],
        out_specs=pl.BlockSpec((BM, D), lambda i: (i, 0)),
        out_shape=jax.ShapeDtypeStruct((N, D), jnp.float32),
    )


_mm_relu = _make_mm(True)
_mm_lin = _make_mm(False)


def kernel(x, edge_index, Wl1, bl1, Wr1, Wl2, bl2, Wr2):
    src = edge_index[0]
    dst = edge_index[1]
    pad = NW * TPW * B - E
    fill = jnp.arange(pad, dtype=jnp.int32)
    src_p = (2 * jnp.concatenate([src, fill % N])).reshape(NW, TPW, B)
    dst_p = jnp.concatenate(
        [dst, N + (fill % (ACC_ROWS - N))]).reshape(NW, TPW, B)
    root1 = _mm_root(x, Wr1, bl1.reshape(1, D))
    p, cnts = _segsum_counts(x.reshape(CF * N, FW), src_p, dst_p)
    h = _mm_relu(p, cnts, root1, Wl1)
    root2 = _mm_root(h, Wr2, bl2.reshape(1, D))
    q = _segsum_plain(h.reshape(CF * N, FW), src_p, dst_p)
    out = _mm_lin(q, cnts, root2, Wl2)
    return out


# speedup vs baseline: 1.0038x; 1.0038x over previous
import functools

import jax
import jax.numpy as jnp
from jax import lax
from jax.experimental import pallas as pl
from jax.experimental.pallas import tpu as pltpu
from jax.experimental.pallas import tpu_sc as plsc

N = 10000
D = 128
E = 320000
NC = 2
NS = 16
NW = NC * NS
B = 128
TPW = 80
ACC_ROWS = 10240
RPT = ACC_ROWS // NS
CNT_W = 16
CF = 2
FW = D // CF
BM = 2000

_MESH = plsc.VectorSubcoreMesh(core_axis_name="c", subcore_axis_name="s")


def _sc_body(with_counts, NBUF, *refs):
    if with_counts:
        (table, src3, dst3, out_p, out_c, src_v, dst_v, *rest) = refs
        rows = rest[:NBUF]
        obuf, zcnt, acc, cnt = rest[NBUF:NBUF + 4]
        sems = rest[NBUF + 4:]
        gs, ss, cs = sems[:NBUF], sems[NBUF:2 * NBUF], sems[2 * NBUF:]
    else:
        table, src3, dst3, out_p, src_v, dst_v, *rest = refs
        rows = rest[:NBUF]
        acc = rest[NBUF]
        sems = rest[NBUF + 1:]
        gs, ss = sems[:NBUF], sems[NBUF:2 * NBUF]
    cid = lax.axis_index("c")
    sid = lax.axis_index("s")
    wid = sid * NC + cid

    pltpu.sync_copy(src3.at[wid], src_v)
    pltpu.sync_copy(dst3.at[wid], dst_v)

    zero16 = jnp.zeros((16,), jnp.float32)
    if with_counts:
        one16 = jnp.ones((16,), jnp.float32)

        def _orow(r, _):
            obuf[r, pl.ds(0, 16)] = one16
            zcnt[r, pl.ds(0, 16)] = zero16
            return 0

        lax.fori_loop(0, B, _orow, 0)

    for cf in range(CF):
        first = cf == 0
        def _zrow(r, _):
            for c8 in range(FW // 16):
                rows[0][r, pl.ds(c8 * 16, 16)] = zero16
            return 0

        lax.fori_loop(0, B, _zrow, 0)
        for k in range(RPT // B):
            pltpu.sync_copy(rows[0], acc.at[pl.ds(sid * RPT + k * B, B)])
            if with_counts and first:
                pltpu.sync_copy(zcnt, cnt.at[pl.ds(sid * RPT + k * B, B)])
        plsc.subcore_barrier()

        wc = with_counts and first

        def _wait_gather(j, b):
            pltpu.make_async_copy(table.at[src_v.at[j]], rows[b],
                                  gs[b]).wait()

        def _issue_scatter(j, b):
            pltpu.async_copy(rows[b], acc.at[dst_v.at[j]], ss[b], add=True)
            if wc:
                pltpu.async_copy(obuf, cnt.at[dst_v.at[j]], cs[b], add=True)

        def _wait_scatter(j, b):
            pltpu.make_async_copy(rows[b], acc.at[dst_v.at[j]], ss[b]).wait()
            if wc:
                pltpu.make_async_copy(obuf, cnt.at[dst_v.at[j]],
                                      cs[b]).wait()

        for b in range(NBUF):
            pltpu.async_copy(table.at[src_v.at[b]], rows[b], gs[b])

        def _super(t, _):
            for b in range(NBUF):
                j = t * NBUF + b
                _wait_gather(j, b)
                _issue_scatter(j, b)
            for b in range(NBUF):
                j = t * NBUF + b
                _wait_scatter(j, b)
                pltpu.async_copy(table.at[src_v.at[j + NBUF]], rows[b],
                                 gs[b])
            return 0

        lax.fori_loop(0, TPW // NBUF - 1, _super, 0)

        last = (TPW // NBUF - 1) * NBUF
        for b in range(NBUF):
            _wait_gather(last + b, b)
            _issue_scatter(last + b, b)
        for b in range(NBUF):
            _wait_scatter(last + b, b)
        plsc.subcore_barrier()

        pltpu.sync_copy(acc.at[pl.ds(sid * RPT, RPT)],
                        out_p.at[cid, pl.ds(sid * RPT, RPT),
                                 pl.ds(cf * FW, FW)])
        if with_counts and first:
            pltpu.sync_copy(cnt.at[pl.ds(sid * RPT, RPT)],
                            out_c.at[cid, pl.ds(sid * RPT, RPT)])

        if first:
            one16i = jnp.ones((16,), jnp.int32)

            def _inc(r, _):
                for c8 in range(B // 16):
                    sl = (r, pl.ds(c8 * 16, 16))
                    src_v[sl] = src_v[sl] + one16i
                return 0

            lax.fori_loop(0, TPW, _inc, 0)


def _make_segsum(with_counts, NBUF):
    out_type = [jax.ShapeDtypeStruct((NC, ACC_ROWS, D), jnp.float32)]
    scratch = [
        pltpu.VMEM((TPW, B), jnp.int32),
        pltpu.VMEM((TPW, B), jnp.int32),
    ]
    scratch += [pltpu.VMEM((B, FW), jnp.float32)] * NBUF
    if with_counts:
        out_type.append(jax.ShapeDtypeStruct((NC, ACC_ROWS, CNT_W),
                                             jnp.float32))
        scratch += [
            pltpu.VMEM((B, CNT_W), jnp.float32),
            pltpu.VMEM((B, CNT_W), jnp.float32),
        ]
    scratch.append(pltpu.VMEM_SHARED((ACC_ROWS, FW), jnp.float32))
    if with_counts:
        scratch.append(pltpu.VMEM_SHARED((ACC_ROWS, CNT_W), jnp.float32))
    n_sems = (3 if with_counts else 2) * NBUF
    scratch += [pltpu.SemaphoreType.DMA] * n_sems
    return pl.kernel(
        functools.partial(_sc_body, with_counts, NBUF),
        out_type=tuple(out_type) if with_counts else out_type[0],
        mesh=_MESH,
        scratch_types=scratch,
        compiler_params=pltpu.CompilerParams(use_tc_tiling_on_sc=False),
    )


_segsum_counts = _make_segsum(True, 4)
_segsum_plain = _make_segsum(False, 8)


def _mm_body(relu, p_ref, c_ref, x_ref, wl_ref, bl_ref, wr_ref, o_ref):
    p = p_ref[0] + p_ref[1]
    cnt = (c_ref[0] + c_ref[1])[:, 0:1]
    mean = p / jnp.maximum(cnt, 1.0)
    y = (jnp.dot(mean, wl_ref[...], preferred_element_type=jnp.float32)
         + bl_ref[...]
         + jnp.dot(x_ref[...], wr_ref[...], preferred_element_type=jnp.float32))
    if relu:
        y = jnp.maximum(y, 0.0)
    o_ref[...] = y


def _make_mm(relu):
    grid = (N // BM,)
    return pl.pallas_call(
        functools.partial(_mm_body, relu),
        grid=grid,
        in_specs=[
            pl.BlockSpec((NC, BM, D), lambda i: (0, i, 0)),
            pl.BlockSpec((NC, BM, CNT_W), lambda i: (0, i, 0)),
            pl.BlockSpec((BM, D), lambda i: (i, 0)),
            pl.BlockSpec((D, D), lambda i: (0, 0)),
            pl.BlockSpec((1, D), lambda i: (0, 0)),
            pl.BlockSpec((D, D), lambda i: (0, 0)),
        ],
        out_specs=pl.BlockSpec((BM, D), lambda i: (i, 0)),
        out_shape=jax.ShapeDtypeStruct((N, D), jnp.float32),
    )


_mm_relu = _make_mm(True)
_mm_lin = _make_mm(False)


def kernel(x, edge_index, Wl1, bl1, Wr1, Wl2, bl2, Wr2):
    src = edge_index[0]
    dst = edge_index[1]
    pad = NW * TPW * B - E
    fill = jnp.arange(pad, dtype=jnp.int32)
    src_p = (2 * jnp.concatenate([src, fill % N])).reshape(NW, TPW, B)
    dst_p = jnp.concatenate(
        [dst, N + (fill % (ACC_ROWS - N))]).reshape(NW, TPW, B)
    p, cnts = _segsum_counts(x.reshape(CF * N, FW), src_p, dst_p)
    h = _mm_relu(p, cnts, x, Wl1, bl1.reshape(1, D), Wr1)
    q = _segsum_plain(h.reshape(CF * N, FW), src_p, dst_p)
    out = _mm_lin(q, cnts, h, Wl2, bl2.reshape(1, D), Wr2)
    return out
